# baseline (speedup 1.0000x reference)
import jax
import jax.numpy as jnp
from jax.experimental import pallas as pl
from jax.experimental.pallas import tpu as pltpu

_TILE = 2048


def _mm_body(m_ref, t_ref, o_ref):
    o_ref[...] = jnp.dot(m_ref[...], t_ref[...],
                         preferred_element_type=jnp.float32)


def kernel(alignments, alignment_embeds):
    B, N, K = alignments.shape
    D = alignment_embeds.shape[-1]
    rows = B * N
    flat = alignments.reshape(rows, K)
    grid = (rows // _TILE,)
    out = pl.pallas_call(
        _mm_body,
        grid=grid,
        in_specs=[
            pl.BlockSpec((_TILE, K), lambda i: (i, 0)),
            pl.BlockSpec((K, D), lambda i: (0, 0)),
        ],
        out_specs=pl.BlockSpec((_TILE, D), lambda i: (i, 0)),
        out_shape=jax.ShapeDtypeStruct((rows, D), jnp.float32),
        compiler_params=pltpu.CompilerParams(
            dimension_semantics=("parallel",),
        ),
    )(flat, alignment_embeds)
    return out.reshape(B, N, D)


# speedup vs baseline: 1.4214x; 1.4214x over previous
import jax
import jax.numpy as jnp
from jax.experimental import pallas as pl
from jax.experimental.pallas import tpu as pltpu

_TILE = 8192


def _mm_body(m_ref, t_ref, o_ref):
    o_ref[...] = jnp.dot(m_ref[...], t_ref[...],
                         preferred_element_type=jnp.float32)


def kernel(alignments, alignment_embeds):
    B, N, K = alignments.shape
    D = alignment_embeds.shape[-1]
    rows = B * N
    flat = alignments.reshape(rows, K)
    grid = (rows // _TILE,)
    out = pl.pallas_call(
        _mm_body,
        grid=grid,
        in_specs=[
            pl.BlockSpec((_TILE, K), lambda i: (i, 0)),
            pl.BlockSpec((K, D), lambda i: (0, 0)),
        ],
        out_specs=pl.BlockSpec((_TILE, D), lambda i: (i, 0)),
        out_shape=jax.ShapeDtypeStruct((rows, D), jnp.float32),
        compiler_params=pltpu.CompilerParams(
            dimension_semantics=("parallel",),
        ),
    )(flat, alignment_embeds)
    return out.reshape(B, N, D)


# speedup vs baseline: 1.4333x; 1.0084x over previous
import jax
import jax.numpy as jnp
from jax.experimental import pallas as pl
from jax.experimental.pallas import tpu as pltpu

_TILE = 16384


def _mm_body(m_ref, t_ref, o_ref):
    o_ref[...] = jnp.dot(m_ref[...], t_ref[...],
                         preferred_element_type=jnp.float32)


def kernel(alignments, alignment_embeds):
    B, N, K = alignments.shape
    D = alignment_embeds.shape[-1]
    rows = B * N
    flat = alignments.reshape(rows, K)
    grid = (rows // _TILE,)
    out = pl.pallas_call(
        _mm_body,
        grid=grid,
        in_specs=[
            pl.BlockSpec((_TILE, K), lambda i: (i, 0)),
            pl.BlockSpec((K, D), lambda i: (0, 0)),
        ],
        out_specs=pl.BlockSpec((_TILE, D), lambda i: (i, 0)),
        out_shape=jax.ShapeDtypeStruct((rows, D), jnp.float32),
        compiler_params=pltpu.CompilerParams(
            dimension_semantics=("parallel",),
        ),
    )(flat, alignment_embeds)
    return out.reshape(B, N, D)
